# Initial kernel scaffold; baseline (speedup 1.0000x reference)
import jax
import jax.numpy as jnp
from jax.experimental import pallas as pl


def kernel(x_num, x_cat, W1, b1, W2, b2, tables):
    raise NotImplementedError("write your pallas kernel here")


# speedup vs baseline: 2.4225x; 2.4225x over previous
import functools

import jax
import jax.numpy as jnp
from jax import lax
from jax.experimental import pallas as pl
from jax.experimental.pallas import tpu as pltpu
from jax.experimental.pallas import tpu_sc as plsc

N_NUM = 100
N_CAT = 26
VOCAB = 1000
EMBED_DIM = 128
BATCH = 16384

NUM_CORES = 2
NUM_SUBCORES = 16
NW = NUM_CORES * NUM_SUBCORES

ROWS = BATCH * N_CAT
BLK = 128
IDX_ROWS = ROWS // BLK
IDX_PER_W = IDX_ROWS // NW


def _sc_gather(tables_flat, idx2d):
    mesh = plsc.VectorSubcoreMesh(core_axis_name="c", subcore_axis_name="s")

    @functools.partial(
        pl.kernel,
        mesh=mesh,
        out_type=jax.ShapeDtypeStruct((ROWS, EMBED_DIM), jnp.float32),
        scratch_types=[
            pltpu.VMEM((BLK,), jnp.int32),
            pltpu.VMEM((BLK, EMBED_DIM), jnp.float32),
            pltpu.SemaphoreType.DMA,
        ],
    )
    def k(tab_hbm, idx_hbm, out_hbm, idxrow_v, buf, gsem):
        wid = lax.axis_index("s") * NUM_CORES + lax.axis_index("c")
        base = wid * IDX_PER_W

        def body(j, carry):
            row = base + j
            pltpu.sync_copy(idx_hbm.at[row], idxrow_v)
            lane = lax.iota(jnp.int32, 16)
            for c in range(BLK // 16):
                r0 = row * BLK + c * 16
                field = lax.rem(r0 + lane, N_CAT)
                idxrow_v[pl.ds(c * 16, 16)] = (
                    field * VOCAB + idxrow_v[pl.ds(c * 16, 16)]
                )
            pltpu.async_copy(tab_hbm.at[idxrow_v], buf, gsem).wait()
            pltpu.sync_copy(buf, out_hbm.at[pl.ds(row * BLK, BLK)])
            return carry

        lax.fori_loop(0, IDX_PER_W, body, 0)

    return k(tables_flat, idx2d)


def _mlp(x_num, W1, b1, W2, b2):
    BM = 1024

    def body(x_ref, w1_ref, b1_ref, w2_ref, b2_ref, o_ref):
        h = jnp.dot(x_ref[...], w1_ref[...],
                    preferred_element_type=jnp.float32) + b1_ref[...]
        h = jnp.maximum(h, 0.0)
        o_ref[...] = jnp.dot(h, w2_ref[...],
                             preferred_element_type=jnp.float32) + b2_ref[...]

    return pl.pallas_call(
        body,
        grid=(BATCH // BM,),
        in_specs=[
            pl.BlockSpec((BM, N_NUM), lambda i: (i, 0)),
            pl.BlockSpec((N_NUM, EMBED_DIM), lambda i: (0, 0)),
            pl.BlockSpec((1, EMBED_DIM), lambda i: (0, 0)),
            pl.BlockSpec((EMBED_DIM, EMBED_DIM), lambda i: (0, 0)),
            pl.BlockSpec((1, EMBED_DIM), lambda i: (0, 0)),
        ],
        out_specs=pl.BlockSpec((BM, EMBED_DIM), lambda i: (i, 0)),
        out_shape=jax.ShapeDtypeStruct((BATCH, EMBED_DIM), jnp.float32),
    )(x_num, W1, b1.reshape(1, EMBED_DIM), W2, b2.reshape(1, EMBED_DIM))


def kernel(x_num, x_cat, W1, b1, W2, b2, tables):
    idx2d = x_cat.astype(jnp.int32).reshape(IDX_ROWS, BLK)
    tables_flat = tables.reshape(N_CAT * VOCAB, EMBED_DIM)
    x_cats = _sc_gather(tables_flat, idx2d).reshape(BATCH, N_CAT, EMBED_DIM)
    num_out = _mlp(x_num, W1, b1, W2, b2)[:, None, :]
    return (num_out, x_cats)


# speedup vs baseline: 2.9462x; 1.2162x over previous
import functools

import jax
import jax.numpy as jnp
from jax import lax
from jax.experimental import pallas as pl
from jax.experimental.pallas import tpu as pltpu
from jax.experimental.pallas import tpu_sc as plsc

N_NUM = 100
N_CAT = 26
VOCAB = 1000
EMBED_DIM = 128
BATCH = 16384

NUM_CORES = 2
NUM_SUBCORES = 16
NW = NUM_CORES * NUM_SUBCORES

ROWS = BATCH * N_CAT
BLK = 128
IDX_ROWS = ROWS // BLK
IDX_PER_W = IDX_ROWS // NW
NBUF = 4


def _sc_gather(tables_flat, idx2d):
    mesh = plsc.VectorSubcoreMesh(core_axis_name="c", subcore_axis_name="s")

    @functools.partial(
        pl.kernel,
        mesh=mesh,
        out_type=jax.ShapeDtypeStruct((ROWS, EMBED_DIM), jnp.float32),
        scratch_types=[
            pltpu.VMEM((IDX_PER_W, BLK), jnp.int32),
            pltpu.VMEM((NBUF, BLK, EMBED_DIM), jnp.float32),
            [pltpu.SemaphoreType.DMA] * NBUF,
            [pltpu.SemaphoreType.DMA] * NBUF,
        ],
    )
    def k(tab_hbm, idx_hbm, out_hbm, idx_v, bufs, gsems, osems):
        wid = lax.axis_index("s") * NUM_CORES + lax.axis_index("c")
        base = wid * IDX_PER_W
        pltpu.sync_copy(idx_hbm.at[pl.ds(base, IDX_PER_W)], idx_v)
        lane = lax.iota(jnp.int32, 16)

        def flats(j):
            for c in range(BLK // 16):
                r0 = (base + j) * BLK + c * 16
                field = lax.rem(r0 + lane, N_CAT)
                idx_v[j, pl.ds(c * 16, 16)] = (
                    field * VOCAB + idx_v[j, pl.ds(c * 16, 16)]
                )

        def gather(j, b):
            return pltpu.make_async_copy(
                tab_hbm.at[idx_v.at[j]], bufs.at[b], gsems[b])

        def out_copy(j, b):
            return pltpu.make_async_copy(
                bufs.at[b], out_hbm.at[pl.ds((base + j) * BLK, BLK)],
                osems[b])

        for b in range(NBUF):
            flats(b)
            gather(b, b).start()

        def step(jo, carry):
            for b in range(NBUF):
                j = jo * NBUF + b
                gather(j, b).wait()
                out_copy(j, b).start()
                jn = j + NBUF

                @pl.when(jn < IDX_PER_W)
                def _():
                    out_copy(j, b).wait()
                    flats(jn)
                    gather(jn, b).start()

            return carry

        lax.fori_loop(0, IDX_PER_W // NBUF, step, 0)
        for b in range(NBUF):
            out_copy(IDX_PER_W - NBUF + b, b).wait()

    return k(tables_flat, idx2d)


def _mlp(x_num, W1, b1, W2, b2):
    BM = 1024

    def body(x_ref, w1_ref, b1_ref, w2_ref, b2_ref, o_ref):
        h = jnp.dot(x_ref[...], w1_ref[...],
                    preferred_element_type=jnp.float32) + b1_ref[...]
        h = jnp.maximum(h, 0.0)
        o_ref[...] = jnp.dot(h, w2_ref[...],
                             preferred_element_type=jnp.float32) + b2_ref[...]

    return pl.pallas_call(
        body,
        grid=(BATCH // BM,),
        in_specs=[
            pl.BlockSpec((BM, N_NUM), lambda i: (i, 0)),
            pl.BlockSpec((N_NUM, EMBED_DIM), lambda i: (0, 0)),
            pl.BlockSpec((1, EMBED_DIM), lambda i: (0, 0)),
            pl.BlockSpec((EMBED_DIM, EMBED_DIM), lambda i: (0, 0)),
            pl.BlockSpec((1, EMBED_DIM), lambda i: (0, 0)),
        ],
        out_specs=pl.BlockSpec((BM, EMBED_DIM), lambda i: (i, 0)),
        out_shape=jax.ShapeDtypeStruct((BATCH, EMBED_DIM), jnp.float32),
    )(x_num, W1, b1.reshape(1, EMBED_DIM), W2, b2.reshape(1, EMBED_DIM))


def kernel(x_num, x_cat, W1, b1, W2, b2, tables):
    idx2d = x_cat.astype(jnp.int32).reshape(IDX_ROWS, BLK)
    tables_flat = tables.reshape(N_CAT * VOCAB, EMBED_DIM)
    x_cats = _sc_gather(tables_flat, idx2d).reshape(BATCH, N_CAT, EMBED_DIM)
    num_out = _mlp(x_num, W1, b1, W2, b2)[:, None, :]
    return (num_out, x_cats)
